# baseline (speedup 1.0000x reference)
import functools

import jax
import jax.numpy as jnp
from jax import lax
from jax.experimental import pallas as pl
from jax.experimental.pallas import tpu as pltpu
from jax.experimental.pallas import tpu_sc as plsc

_SLOPE = (1.0 / 8.0 + 1.0 / 3.0) / 2.0


def _sc_gather_sum(features, idx3, nw, nsub, s_nodes, k_samp, feat_dim):
    b_total = nw * nsub * s_nodes
    rows_per_sub = s_nodes * k_samp
    npw = nsub * s_nodes
    nring = 4
    npair = nsub // 2
    onodes = nring * 2 * s_nodes
    ncc = npw // onodes

    info = plsc.get_sparse_core_info()
    nc = info.num_cores

    mesh = plsc.VectorSubcoreMesh(core_axis_name="c", subcore_axis_name="s")

    rows_t = pltpu.VMEM((2 * rows_per_sub, feat_dim), jnp.float32)
    agg_t = pltpu.VMEM((onodes, feat_dim), jnp.float32)

    @functools.partial(
        pl.kernel,
        mesh=mesh,
        out_type=jax.ShapeDtypeStruct((b_total, feat_dim), jnp.float32),
        scratch_types=[pltpu.VMEM((nsub, rows_per_sub), jnp.int32)]
        + [rows_t] * nring
        + [agg_t, agg_t, pltpu.SemaphoreType.DMA, pltpu.SemaphoreType.DMA],
    )
    def k(table_hbm, idx_hbm, out_hbm, idx_v, *rest):
        rows = rest[:nring]
        agg = rest[nring : nring + 2]
        sem, osem = rest[nring + 2 :]
        wid = lax.axis_index("s") * nc + lax.axis_index("c")
        obase = wid * npw
        pltpu.sync_copy(idx_hbm.at[wid], idx_v)

        for r in range(nring):
            for h in range(2):
                pltpu.async_copy(
                    table_hbm.at[idx_v.at[2 * r + h]],
                    rows[r].at[pl.ds(h * rows_per_sub, rows_per_sub)],
                    sem,
                )

        def group(cc, carry):
            for p in range(2):
                cc2 = cc * 2 + p
                agg_v = agg[p]
                @pl.when(cc2 >= 2)
                def _():
                    pltpu.make_async_copy(
                        agg_v, out_hbm.at[pl.ds(obase, onodes)], osem
                    ).wait()

                for r in range(nring):
                    pr = cc2 * nring + r
                    rows_v = rows[r]
                    pltpu.make_async_copy(
                        table_hbm.at[idx_v.at[0]], rows_v, sem
                    ).wait()

                    @plsc.parallel_loop(0, 2 * s_nodes, unroll=8)
                    def node(n):
                        base = n * k_samp
                        for g in range(feat_dim // 16):
                            sl = pl.ds(g * 16, 16)
                            a0 = rows_v[base, sl] + rows_v[base + 1, sl]
                            a1 = rows_v[base + 2, sl] + rows_v[base + 3, sl]
                            a2 = rows_v[base + 4, sl] + rows_v[base + 5, sl]
                            a3 = rows_v[base + 6, sl] + rows_v[base + 7, sl]
                            a0 = a0 + a1
                            a2 = a2 + a3
                            for j in range(8, k_samp):
                                a0 = a0 + rows_v[base + j, sl]
                            agg_v[r * 2 * s_nodes + n, sl] = a0 + a2
                    pnext = pr + nring
                    nxt = jnp.where(pnext < npair, pnext, pnext - npair)
                    for h in range(2):
                        pltpu.async_copy(
                            table_hbm.at[idx_v.at[2 * nxt + h]],
                            rows_v.at[pl.ds(h * rows_per_sub, rows_per_sub)],
                            sem,
                        )

                pltpu.async_copy(
                    agg_v, out_hbm.at[pl.ds(obase + cc2 * onodes, onodes)], osem
                )
            return carry

        lax.fori_loop(0, ncc // 2, group, 0)
        for r in range(nring):
            pltpu.make_async_copy(
                table_hbm.at[idx_v.at[0]], rows[r], sem
            ).wait()
        for p in range(2):
            pltpu.make_async_copy(
                agg[p], out_hbm.at[pl.ds(obase, onodes)], osem
            ).wait()

    return k(features, idx3)


def _tc_project(sums, W, block_b):
    b_total, feat_dim = sums.shape
    embed_dim = W.shape[0]

    def body(w_ref, a_ref, o_ref):
        a = a_ref[...]
        w = w_ref[...]
        pre = lax.dot_general(
            w, a, (((1,), (1,)), ((), ())), preferred_element_type=jnp.float32
        ) * jnp.float32(0.1)
        o_ref[...] = jnp.where(pre >= 0, pre, jnp.float32(_SLOPE) * pre)

    return pl.pallas_call(
        body,
        grid=(b_total // block_b,),
        in_specs=[
            pl.BlockSpec((embed_dim, feat_dim), lambda i: (0, 0)),
            pl.BlockSpec((block_b, feat_dim), lambda i: (i, 0)),
        ],
        out_specs=pl.BlockSpec((embed_dim, block_b), lambda i: (0, i)),
        out_shape=jax.ShapeDtypeStruct((embed_dim, b_total), jnp.float32),
    )(W, sums)


def kernel(nodes, features, neigh_idx, W):
    del nodes
    batch, k_samp = neigh_idx.shape
    feat_dim = features.shape[1]

    nw = 32
    s_nodes = 8
    nsub = batch // (nw * s_nodes)
    idx3 = neigh_idx.reshape(nw, nsub, s_nodes * k_samp)

    sums = _sc_gather_sum(features, idx3, nw, nsub, s_nodes, k_samp, feat_dim)
    return _tc_project(sums, W, 4096)


# speedup vs baseline: 1.2594x; 1.2594x over previous
import functools

import jax
import jax.numpy as jnp
from jax import lax
from jax.experimental import pallas as pl
from jax.experimental.pallas import tpu as pltpu
from jax.experimental.pallas import tpu_sc as plsc

_SLOPE = (1.0 / 8.0 + 1.0 / 3.0) / 2.0


def _sc_gather_sum(features, idx3, nw, nsub, s_nodes, k_samp, feat_dim):
    b_total = nw * nsub * s_nodes
    rows_per_sub = s_nodes * k_samp
    npw = nsub * s_nodes
    nring = 4
    npair = nsub // 2
    onodes = nring * 2 * s_nodes
    ncc = npw // onodes

    info = plsc.get_sparse_core_info()
    nc = info.num_cores

    mesh = plsc.VectorSubcoreMesh(core_axis_name="c", subcore_axis_name="s")

    rows_t = pltpu.VMEM((2 * rows_per_sub, feat_dim), jnp.float32)
    agg_t = pltpu.VMEM((onodes, feat_dim), jnp.float32)

    @functools.partial(
        pl.kernel,
        mesh=mesh,
        out_type=jax.ShapeDtypeStruct((b_total, feat_dim), jnp.float32),
        scratch_types=[pltpu.VMEM((nsub, rows_per_sub), jnp.int32)]
        + [rows_t] * nring
        + [agg_t, agg_t, pltpu.SemaphoreType.DMA, pltpu.SemaphoreType.DMA],
    )
    def k(table_hbm, idx_hbm, out_hbm, idx_v, *rest):
        rows = rest[:nring]
        agg = rest[nring : nring + 2]
        sem, osem = rest[nring + 2 :]
        wid = lax.axis_index("s") * nc + lax.axis_index("c")
        obase = wid * npw
        pltpu.sync_copy(idx_hbm.at[wid], idx_v)

        for r in range(nring):
            for h in range(2):
                pltpu.async_copy(
                    table_hbm.at[idx_v.at[2 * r + h]],
                    rows[r].at[pl.ds(h * rows_per_sub, rows_per_sub)],
                    sem,
                )

        def group(cc, carry):
            for p in range(2):
                cc2 = cc * 2 + p
                agg_v = agg[p]
                @pl.when(cc2 >= 2)
                def _():
                    pltpu.make_async_copy(
                        agg_v, out_hbm.at[pl.ds(obase, onodes)], osem
                    ).wait()

                for r in range(nring):
                    pr = cc2 * nring + r
                    rows_v = rows[r]
                    pltpu.make_async_copy(
                        table_hbm.at[idx_v.at[0]], rows_v, sem
                    ).wait()

                    @plsc.parallel_loop(0, 2 * s_nodes, unroll=4)
                    def node(n):
                        base = n * k_samp
                        for g in range(feat_dim // 16):
                            sl = pl.ds(g * 16, 16)
                            a0 = rows_v[base, sl] + rows_v[base + 1, sl]
                            a1 = rows_v[base + 2, sl] + rows_v[base + 3, sl]
                            a2 = rows_v[base + 4, sl] + rows_v[base + 5, sl]
                            a3 = rows_v[base + 6, sl] + rows_v[base + 7, sl]
                            a0 = a0 + a1
                            a2 = a2 + a3
                            for j in range(8, k_samp):
                                a0 = a0 + rows_v[base + j, sl]
                            agg_v[r * 2 * s_nodes + n, sl] = a0 + a2
                    pnext = pr + nring
                    nxt = jnp.where(pnext < npair, pnext, pnext - npair)
                    for h in range(2):
                        pltpu.async_copy(
                            table_hbm.at[idx_v.at[2 * nxt + h]],
                            rows_v.at[pl.ds(h * rows_per_sub, rows_per_sub)],
                            sem,
                        )

                pltpu.async_copy(
                    agg_v, out_hbm.at[pl.ds(obase + cc2 * onodes, onodes)], osem
                )
            return carry

        lax.fori_loop(0, ncc // 2, group, 0)
        for r in range(nring):
            pltpu.make_async_copy(
                table_hbm.at[idx_v.at[0]], rows[r], sem
            ).wait()
        for p in range(2):
            pltpu.make_async_copy(
                agg[p], out_hbm.at[pl.ds(obase, onodes)], osem
            ).wait()

    return k(features, idx3)


def _tc_project(sums, W, block_b):
    b_total, feat_dim = sums.shape
    embed_dim = W.shape[0]

    def body(w_ref, a_ref, o_ref):
        a = a_ref[...]
        w = w_ref[...]
        pre = lax.dot_general(
            w, a, (((1,), (1,)), ((), ())), preferred_element_type=jnp.float32
        ) * jnp.float32(0.1)
        o_ref[...] = jnp.where(pre >= 0, pre, jnp.float32(_SLOPE) * pre)

    return pl.pallas_call(
        body,
        grid=(b_total // block_b,),
        in_specs=[
            pl.BlockSpec((embed_dim, feat_dim), lambda i: (0, 0)),
            pl.BlockSpec((block_b, feat_dim), lambda i: (i, 0)),
        ],
        out_specs=pl.BlockSpec((embed_dim, block_b), lambda i: (0, i)),
        out_shape=jax.ShapeDtypeStruct((embed_dim, b_total), jnp.float32),
    )(W, sums)


def kernel(nodes, features, neigh_idx, W):
    del nodes
    batch, k_samp = neigh_idx.shape
    feat_dim = features.shape[1]

    nw = 32
    s_nodes = 8
    nsub = batch // (nw * s_nodes)
    idx3 = neigh_idx.reshape(nw, nsub, s_nodes * k_samp)

    sums = _sc_gather_sum(features, idx3, nw, nsub, s_nodes, k_samp, feat_dim)
    return _tc_project(sums, W, 8192)
